# baseline (speedup 1.0000x reference)
import functools

import jax
import jax.numpy as jnp
from jax import lax
from jax.experimental import pallas as pl
from jax.experimental.pallas import tpu as pltpu
from jax.experimental.pallas import tpu_sc as plsc

N = 1048576
D = 3
T = 2097152
F = 8
RES = 101

P1 = -1640531535
P2 = 805459861

NW = 32
PTS = N // NW
P = 512
NG = P // 16
NCHUNK = PTS // P


def _iota16():
    return lax.broadcasted_iota(jnp.int32, (16,), 0)


def _full16(v):
    return jnp.full((16,), v, jnp.int32)


_mesh = plsc.VectorSubcoreMesh(core_axis_name="c", subcore_axis_name="s")


@functools.partial(
    pl.kernel,
    mesh=_mesh,
    compiler_params=pltpu.CompilerParams(use_tc_tiling_on_sc=False,
                                         needs_layout_passes=False),
    out_type=jax.ShapeDtypeStruct((N * F,), jnp.float32),
    scratch_types=[
        pltpu.VMEM((3, P), jnp.float32),
        pltpu.VMEM((NG, 128), jnp.int32),
        pltpu.VMEM((NG, 128), jnp.int32),
        pltpu.VMEM((8 * P, 16), jnp.float32),
        pltpu.VMEM((P * F,), jnp.float32),
        pltpu.VMEM((3, P), jnp.float32),
        pltpu.SemaphoreType.DMA,
    ],
)
def _grid_lookup(xt_hbm, table_hbm, out_hbm, wbuf, idxbuf, pbuf, rows, obuf,
                 xbuf, gsem):
    i32 = jnp.int32
    wid = lax.axis_index("s") * i32(2) + lax.axis_index("c")
    base = wid * i32(PTS)
    iot = _iota16()

    def chunk_body(t, carry):
        cbase = base + t * i32(P)
        pltpu.sync_copy(xt_hbm.at[:, pl.ds(cbase, P)], xbuf)

        def hash_group(g, c2):
            off = g * i32(16)
            ints = []
            for d in range(3):
                xs = (xbuf[d, pl.ds(off, 16)] + 1.0) / 2.0 * (RES - 1)
                ii = xs.astype(jnp.int32)
                wbuf[d, pl.ds(off, 16)] = xs - ii.astype(jnp.float32)
                ints.append(ii)
            ix, iy, iz = ints
            a0 = ix
            a1 = ix + 1
            b0 = iy * P1
            b1 = b0 + P1
            c0 = iz * P2
            c1 = c0 + P2
            for c in range(8):
                h = (a1 if c & 4 else a0) ^ (b1 if c & 2 else b0)
                h = (h ^ (c1 if c & 1 else c0)) & (T - 1)
                idxbuf[g, pl.ds(c * 16, 16)] = lax.shift_right_logical(h, jnp.int32(1))
                pbuf[g, pl.ds(c * 16, 16)] = lax.shift_left(h & 1, jnp.int32(3))
            pltpu.async_copy(table_hbm.at[idxbuf.at[g]],
                             rows.at[pl.ds(g * i32(128), 128)], gsem)
            return c2

        lax.fori_loop(i32(0), i32(NG), hash_group, i32(0))
        pltpu.make_async_copy(table_hbm.at[pl.ds(0, 8 * P)], rows, gsem).wait()

        def interp_group(g, c2):
            off = g * i32(16)
            wx = wbuf[0, pl.ds(off, 16)]
            wy = wbuf[1, pl.ds(off, 16)]
            wz = wbuf[2, pl.ds(off, 16)]
            ux = 1.0 - wx
            uy = 1.0 - wy
            uz = 1.0 - wz
            e00 = ux * uy
            e01 = ux * wy
            e10 = wx * uy
            e11 = wx * wy
            exy = [e00, e01, e10, e11]
            accs = [jnp.zeros((16,), jnp.float32) for _ in range(F)]
            rowbase = g * 128
            for c in range(8):
                wc = exy[c >> 1] * (wz if c & 1 else uz)
                ridx = _full16(rowbase + c * 16) + iot
                par = pbuf[g, pl.ds(c * 16, 16)]
                for f in range(F):
                    v = plsc.load_gather(rows, [ridx, par + f])
                    accs[f] = accs[f] + wc * v
            pidx = (_full16(off) + iot) * 8
            for f in range(F):
                plsc.store_scatter(obuf, [pidx + f], accs[f])
            return c2

        lax.fori_loop(i32(0), i32(NG), interp_group, i32(0))
        pltpu.sync_copy(obuf, out_hbm.at[pl.ds(cbase * 8, P * F)])
        return carry

    lax.fori_loop(i32(0), i32(NCHUNK), chunk_body, i32(0))


def kernel(X, hash_table):
    xt = X.astype(jnp.float32).T
    table2 = hash_table.astype(jnp.float32).reshape(T // 2, 16)
    return _grid_lookup(xt, table2).reshape(N, F)


# speedup vs baseline: 1.4603x; 1.4603x over previous
import functools

import jax
import jax.numpy as jnp
from jax import lax
from jax.experimental import pallas as pl
from jax.experimental.pallas import tpu as pltpu
from jax.experimental.pallas import tpu_sc as plsc

N = 1048576
D = 3
T = 2097152
F = 8
RES = 101

P1 = -1640531535
P2 = 805459861

NW = 32
PTS = N // NW
P = 512
NG = P // 16
NCHUNK = PTS // P


def _iota16():
    return lax.broadcasted_iota(jnp.int32, (16,), 0)


def _full16(v):
    return jnp.full((16,), v, jnp.int32)


_mesh = plsc.VectorSubcoreMesh(core_axis_name="c", subcore_axis_name="s")


@functools.partial(
    pl.kernel,
    mesh=_mesh,
    compiler_params=pltpu.CompilerParams(use_tc_tiling_on_sc=False,
                                         needs_layout_passes=False),
    out_type=jax.ShapeDtypeStruct((N * F,), jnp.float32),
    scratch_types=[
        pltpu.VMEM((3, P), jnp.float32),
        pltpu.VMEM((NG, 128), jnp.int32),
        pltpu.VMEM((8 * P, F), jnp.float32),
        pltpu.VMEM((P * F,), jnp.float32),
        pltpu.VMEM((3, P), jnp.float32),
        pltpu.SemaphoreType.DMA,
    ],
)
def _grid_lookup(xt_hbm, table_hbm, out_hbm, wbuf, idxbuf, rows, obuf, xbuf,
                 gsem):
    i32 = jnp.int32
    wid = lax.axis_index("s") * i32(2) + lax.axis_index("c")
    base = wid * i32(PTS)
    iot = _iota16()

    def chunk_body(t, carry):
        cbase = base + t * i32(P)
        pltpu.sync_copy(xt_hbm.at[:, pl.ds(cbase, P)], xbuf)

        def hash_group(g, c2):
            off = g * i32(16)
            ints = []
            for d in range(3):
                xs = (xbuf[d, pl.ds(off, 16)] + 1.0) / 2.0 * (RES - 1)
                ii = xs.astype(jnp.int32)
                wbuf[d, pl.ds(off, 16)] = xs - ii.astype(jnp.float32)
                ints.append(ii)
            ix, iy, iz = ints
            a0 = ix
            a1 = ix + 1
            b0 = iy * P1
            b1 = b0 + P1
            c0 = iz * P2
            c1 = c0 + P2
            for c in range(8):
                h = (a1 if c & 4 else a0) ^ (b1 if c & 2 else b0)
                h = (h ^ (c1 if c & 1 else c0)) & (T - 1)
                idxbuf[g, pl.ds(c * 16, 16)] = h
            pltpu.async_copy(table_hbm.at[idxbuf.at[g]],
                             rows.at[pl.ds(g * i32(128), 128)], gsem)
            return c2

        lax.fori_loop(i32(0), i32(NG), hash_group, i32(0))
        pltpu.make_async_copy(table_hbm.at[pl.ds(0, 8 * P)], rows, gsem).wait()

        def interp_group(g, c2):
            off = g * i32(16)
            wx = wbuf[0, pl.ds(off, 16)]
            wy = wbuf[1, pl.ds(off, 16)]
            wz = wbuf[2, pl.ds(off, 16)]
            ux = 1.0 - wx
            uy = 1.0 - wy
            uz = 1.0 - wz
            e00 = ux * uy
            e01 = ux * wy
            e10 = wx * uy
            e11 = wx * wy
            exy = [e00, e01, e10, e11]
            accs = [jnp.zeros((16,), jnp.float32) for _ in range(F)]
            rowbase = g * 128
            for c in range(8):
                wc = exy[c >> 1] * (wz if c & 1 else uz)
                ridx = _full16(rowbase + c * 16) + iot
                for f in range(F):
                    v = plsc.load_gather(rows, [ridx, _full16(f)])
                    accs[f] = accs[f] + wc * v
            obase = lax.div(g, i32(8)) * i32(1024) + lax.rem(g, i32(8)) * i32(16)
            for f in range(F):
                obuf[pl.ds(obase + f * 128, 16)] = accs[f]
            return c2

        lax.fori_loop(i32(0), i32(NG), interp_group, i32(0))
        pltpu.sync_copy(obuf, out_hbm.at[pl.ds(cbase * 8, P * F)])
        return carry

    lax.fori_loop(i32(0), i32(NCHUNK), chunk_body, i32(0))


def kernel(X, hash_table):
    xt = X.astype(jnp.float32).T
    o = _grid_lookup(xt, hash_table.astype(jnp.float32))
    return o.reshape(N // 128, F, 128).swapaxes(1, 2).reshape(N, F)


# speedup vs baseline: 2.7042x; 1.8518x over previous
import functools

import jax
import jax.numpy as jnp
from jax import lax
from jax.experimental import pallas as pl
from jax.experimental.pallas import tpu as pltpu
from jax.experimental.pallas import tpu_sc as plsc

N = 1048576
D = 3
T = 2097152
F = 8
RES = 101

P1 = -1640531535
P2 = 805459861

NW = 32
PTS = N // NW
P = 512
NG = P // 16
NCHUNK = PTS // P

NBLK = T // 128
BPW = NBLK // NW
BB = 16


def _iota16():
    return lax.broadcasted_iota(jnp.int32, (16,), 0)


def _full16(v):
    return jnp.full((16,), v, jnp.int32)


_mesh = plsc.VectorSubcoreMesh(core_axis_name="c", subcore_axis_name="s")
_params = pltpu.CompilerParams(use_tc_tiling_on_sc=False,
                               needs_layout_passes=False)


@functools.partial(
    pl.kernel,
    mesh=_mesh,
    compiler_params=_params,
    out_type=jax.ShapeDtypeStruct((T * F,), jnp.float32),
    scratch_types=[
        pltpu.VMEM((BB, F, 128), jnp.float32),
        pltpu.VMEM((BB * 128 * F,), jnp.float32),
    ],
)
def _table_rows(tv_hbm, out_hbm, inbuf, outbuf):
    i32 = jnp.int32
    wid = lax.axis_index("s") * i32(2) + lax.axis_index("c")
    iot8 = _iota16() * 8

    def batch_body(bi, carry):
        bb = wid * i32(BPW) + bi * i32(BB)
        pltpu.sync_copy(tv_hbm.at[pl.ds(bb, BB)], inbuf)

        def blk_body(blk, c2):
            for f in range(F):
                for j in range(8):
                    v = inbuf[blk, f, pl.ds(j * 16, 16)]
                    dst = _full16(blk * i32(1024) + i32(j * 128 + f)) + iot8
                    plsc.store_scatter(outbuf, [dst], v)
            return c2

        lax.fori_loop(i32(0), i32(BB), blk_body, i32(0))
        pltpu.sync_copy(outbuf, out_hbm.at[pl.ds(bb * 1024, BB * 1024)])
        return carry

    lax.fori_loop(i32(0), i32(BPW // BB), batch_body, i32(0))


@functools.partial(
    pl.kernel,
    mesh=_mesh,
    compiler_params=_params,
    out_type=jax.ShapeDtypeStruct((N * F,), jnp.float32),
    scratch_types=[
        pltpu.VMEM((3, P), jnp.float32),
        pltpu.VMEM((NG, 128), jnp.int32),
        pltpu.VMEM((8 * P, F), jnp.float32),
        pltpu.VMEM((P * F,), jnp.float32),
        pltpu.VMEM((3, P), jnp.float32),
        pltpu.SemaphoreType.DMA,
    ],
)
def _grid_lookup(xt_hbm, table_hbm, out_hbm, wbuf, idxbuf, rows, obuf, xbuf,
                 gsem):
    i32 = jnp.int32
    wid = lax.axis_index("s") * i32(2) + lax.axis_index("c")
    base = wid * i32(PTS)
    iot = _iota16()

    def chunk_body(t, carry):
        cbase = base + t * i32(P)
        pltpu.sync_copy(xt_hbm.at[:, pl.ds(cbase, P)], xbuf)

        def hash_group(g, c2):
            off = g * i32(16)
            ints = []
            for d in range(3):
                xs = (xbuf[d, pl.ds(off, 16)] + 1.0) / 2.0 * (RES - 1)
                ii = xs.astype(jnp.int32)
                wbuf[d, pl.ds(off, 16)] = xs - ii.astype(jnp.float32)
                ints.append(ii)
            ix, iy, iz = ints
            a0 = ix
            a1 = ix + 1
            b0 = iy * P1
            b1 = b0 + P1
            c0 = iz * P2
            c1 = c0 + P2
            for c in range(8):
                h = (a1 if c & 4 else a0) ^ (b1 if c & 2 else b0)
                h = (h ^ (c1 if c & 1 else c0)) & (T - 1)
                idxbuf[g, pl.ds(c * 16, 16)] = h
            pltpu.async_copy(table_hbm.at[idxbuf.at[g]],
                             rows.at[pl.ds(g * i32(128), 128)], gsem)
            return c2

        lax.fori_loop(i32(0), i32(NG), hash_group, i32(0))
        pltpu.make_async_copy(table_hbm.at[pl.ds(0, 8 * P)], rows, gsem).wait()

        def interp_group(g, c2):
            off = g * i32(16)
            wx = wbuf[0, pl.ds(off, 16)]
            wy = wbuf[1, pl.ds(off, 16)]
            wz = wbuf[2, pl.ds(off, 16)]
            ux = 1.0 - wx
            uy = 1.0 - wy
            uz = 1.0 - wz
            e00 = ux * uy
            e01 = ux * wy
            e10 = wx * uy
            e11 = wx * wy
            exy = [e00, e01, e10, e11]
            accs = [jnp.zeros((16,), jnp.float32) for _ in range(F)]
            rowbase = g * 128
            for c in range(8):
                wc = exy[c >> 1] * (wz if c & 1 else uz)
                ridx = _full16(rowbase + c * 16) + iot
                for f in range(F):
                    v = plsc.load_gather(rows, [ridx, _full16(f)])
                    accs[f] = accs[f] + wc * v
            obase = lax.div(g, i32(8)) * i32(1024) + lax.rem(g, i32(8)) * i32(16)
            for f in range(F):
                obuf[pl.ds(obase + f * 128, 16)] = accs[f]
            return c2

        lax.fori_loop(i32(0), i32(NG), interp_group, i32(0))
        pltpu.sync_copy(obuf, out_hbm.at[pl.ds(cbase * 8, P * F)])
        return carry

    lax.fori_loop(i32(0), i32(NCHUNK), chunk_body, i32(0))


def kernel(X, hash_table):
    xt = X.astype(jnp.float32).T
    tv = (hash_table.astype(jnp.float32)
          .reshape(T // 128, 128, F).swapaxes(1, 2))
    rows_flat = _table_rows(tv)
    o = _grid_lookup(xt, rows_flat.reshape(T, F))
    return o.reshape(N // 128, F, 128).swapaxes(1, 2).reshape(N, F)


# speedup vs baseline: 4.7337x; 1.7505x over previous
import functools

import jax
import jax.numpy as jnp
from jax import lax
from jax.experimental import pallas as pl
from jax.experimental.pallas import tpu as pltpu
from jax.experimental.pallas import tpu_sc as plsc

N = 1048576
D = 3
T = 2097152
F = 8
RES = 101

P1 = -1640531535
P2 = 805459861

NW = 32
PTS = N // NW
P = 512
NG = P // 16
NCHUNK = PTS // P
NPAIR = NCHUNK // 2

NBLK = T // 128
BPW = NBLK // NW
BB = 16
NBATCH = BPW // BB


def _iota16():
    return lax.broadcasted_iota(jnp.int32, (16,), 0)


def _full16(v):
    return jnp.full((16,), v, jnp.int32)


_mesh = plsc.VectorSubcoreMesh(core_axis_name="c", subcore_axis_name="s")
_params = pltpu.CompilerParams(use_tc_tiling_on_sc=False,
                               needs_layout_passes=False)


@functools.partial(
    pl.kernel,
    mesh=_mesh,
    compiler_params=_params,
    out_type=jax.ShapeDtypeStruct((T * F,), jnp.float32),
    scratch_types=[
        pltpu.VMEM((BB, F, 128), jnp.float32),
        pltpu.VMEM((BB, F, 128), jnp.float32),
        pltpu.VMEM((BB * 128 * F,), jnp.float32),
        pltpu.VMEM((BB * 128 * F,), jnp.float32),
        pltpu.SemaphoreType.DMA,
        pltpu.SemaphoreType.DMA,
        pltpu.SemaphoreType.DMA,
        pltpu.SemaphoreType.DMA,
    ],
)
def _table_rows(tv_hbm, out_hbm, in0, in1, ou0, ou1, is0, is1, os0, os1):
    i32 = jnp.int32
    wid = lax.axis_index("s") * i32(2) + lax.axis_index("c")
    wbase = wid * i32(BPW)
    iot = _iota16()
    fpat = []
    cpat = []
    for k in range(8):
        m = iot + k * 16
        fpat.append(m & 7)
        cpat.append(lax.shift_right_logical(m, jnp.int32(3)))

    ins = (in0, in1)
    ous = (ou0, ou1)
    isems = (is0, is1)
    osems = (os0, os1)

    def transpose_batch(ib, ob):
        def blk_body(blk, c2):
            bv = _full16(blk)
            for k in range(8):
                v = plsc.load_gather(ib, [bv, fpat[k], cpat[k]])
                ob[pl.ds(blk * i32(1024) + i32(k * 16), 16)] = v
            return c2

        lax.fori_loop(i32(0), i32(BB), blk_body, i32(0))

    def fire_in(bi, s):
        pltpu.async_copy(tv_hbm.at[pl.ds(wbase + bi * i32(BB), BB)],
                         ins[s], isems[s])

    def wait_in(s):
        pltpu.make_async_copy(tv_hbm.at[pl.ds(0, BB)], ins[s],
                              isems[s]).wait()

    def fire_out(bi, s):
        pltpu.async_copy(ous[s],
                         out_hbm.at[pl.ds((wbase + bi * i32(BB)) * i32(1024),
                                          BB * 1024)],
                         osems[s])

    def wait_out(s):
        pltpu.make_async_copy(ous[s], out_hbm.at[pl.ds(0, BB * 1024)],
                              osems[s]).wait()

    fire_in(i32(0), 0)

    def pair_body(t2, carry):
        bi0 = t2 * i32(2)
        fire_in(bi0 + 1, 1)
        wait_in(0)

        @pl.when(t2 > 0)
        def _():
            wait_out(0)

        transpose_batch(ins[0], ous[0])
        fire_out(bi0, 0)

        @pl.when(bi0 + 2 < NBATCH)
        def _():
            fire_in(bi0 + 2, 0)

        wait_in(1)

        @pl.when(t2 > 0)
        def _():
            wait_out(1)

        transpose_batch(ins[1], ous[1])
        fire_out(bi0 + 1, 1)
        return carry

    lax.fori_loop(i32(0), i32(NBATCH // 2), pair_body, i32(0))
    wait_out(0)
    wait_out(1)


@functools.partial(
    pl.kernel,
    mesh=_mesh,
    compiler_params=_params,
    out_type=jax.ShapeDtypeStruct((N * F,), jnp.float32),
    scratch_types=[
        pltpu.VMEM((3, P), jnp.float32),
        pltpu.VMEM((3, P), jnp.float32),
        pltpu.VMEM((NG, 128), jnp.int32),
        pltpu.VMEM((NG, 128), jnp.int32),
        pltpu.VMEM((8 * P, F), jnp.float32),
        pltpu.VMEM((8 * P, F), jnp.float32),
        pltpu.VMEM((P * F,), jnp.float32),
        pltpu.VMEM((P * F,), jnp.float32),
        pltpu.VMEM((3, 2 * P), jnp.float32),
        pltpu.SemaphoreType.DMA,
        pltpu.SemaphoreType.DMA,
        pltpu.SemaphoreType.DMA,
        pltpu.SemaphoreType.DMA,
    ],
)
def _grid_lookup(xt_hbm, table_hbm, out_hbm, wb0, wb1, ib0, ib1, rb0, rb1,
                 ob0, ob1, xb, g0, g1, o0, o1):
    i32 = jnp.int32
    wid = lax.axis_index("s") * i32(2) + lax.axis_index("c")
    base = wid * i32(PTS)
    iot = _iota16()

    wbs = (wb0, wb1)
    ibs = (ib0, ib1)
    rbs = (rb0, rb1)
    obs = (ob0, ob1)
    gsems = (g0, g1)
    osems = (o0, o1)

    def load_x_pair(t2):
        pltpu.sync_copy(xt_hbm.at[:, pl.ds(base + t2 * i32(2 * P), 2 * P)], xb)

    def fire_chunk(xoff, s):
        wb, ib, rb, gsem = wbs[s], ibs[s], rbs[s], gsems[s]

        def hash_group(g, c2):
            off = g * i32(16)
            ints = []
            for d in range(3):
                xs = (xb[d, pl.ds(xoff + off, 16)] + 1.0) / 2.0 * (RES - 1)
                ii = xs.astype(jnp.int32)
                wb[d, pl.ds(off, 16)] = xs - ii.astype(jnp.float32)
                ints.append(ii)
            ix, iy, iz = ints
            a0 = ix
            a1 = ix + 1
            b0 = iy * P1
            b1 = b0 + P1
            c0 = iz * P2
            c1 = c0 + P2
            for c in range(8):
                h = (a1 if c & 4 else a0) ^ (b1 if c & 2 else b0)
                h = (h ^ (c1 if c & 1 else c0)) & (T - 1)
                ib[g, pl.ds(c * 16, 16)] = h
            pltpu.async_copy(table_hbm.at[ib.at[g]],
                             rb.at[pl.ds(g * i32(128), 128)], gsem)
            return c2

        lax.fori_loop(i32(0), i32(NG), hash_group, i32(0))

    def drain_chunk(s):
        pltpu.make_async_copy(table_hbm.at[pl.ds(0, 8 * P)], rbs[s],
                              gsems[s]).wait()

    def wait_out(s):
        pltpu.make_async_copy(obs[s], out_hbm.at[pl.ds(0, P * F)],
                              osems[s]).wait()

    def interp_chunk(tc, s):
        wb, rb, ob, osem = wbs[s], rbs[s], obs[s], osems[s]

        def interp_group(g, c2):
            off = g * i32(16)
            wx = wb[0, pl.ds(off, 16)]
            wy = wb[1, pl.ds(off, 16)]
            wz = wb[2, pl.ds(off, 16)]
            ux = 1.0 - wx
            uy = 1.0 - wy
            uz = 1.0 - wz
            exy = [ux * uy, ux * wy, wx * uy, wx * wy]
            accs = [jnp.zeros((16,), jnp.float32) for _ in range(F)]
            rowbase = g * 128
            for c in range(8):
                wc = exy[c >> 1] * (wz if c & 1 else uz)
                ridx = _full16(rowbase + c * 16) + iot
                for f in range(F):
                    v = plsc.load_gather(rb, [ridx, _full16(f)])
                    accs[f] = accs[f] + wc * v
            obase = (lax.div(g, i32(8)) * i32(1024)
                     + lax.rem(g, i32(8)) * i32(16))
            for f in range(F):
                ob[pl.ds(obase + f * 128, 16)] = accs[f]
            return c2

        lax.fori_loop(i32(0), i32(NG), interp_group, i32(0))
        pltpu.async_copy(ob, out_hbm.at[pl.ds((base + tc * i32(P)) * 8,
                                              P * F)], osem)

    load_x_pair(i32(0))
    fire_chunk(i32(0), 0)

    def pair_body(t2, carry):
        tc0 = t2 * i32(2)
        fire_chunk(i32(P), 1)
        drain_chunk(0)

        @pl.when(t2 > 0)
        def _():
            wait_out(0)

        interp_chunk(tc0, 0)

        @pl.when(t2 + 1 < NPAIR)
        def _():
            load_x_pair(t2 + 1)
            fire_chunk(i32(0), 0)

        drain_chunk(1)

        @pl.when(t2 > 0)
        def _():
            wait_out(1)

        interp_chunk(tc0 + 1, 1)
        return carry

    lax.fori_loop(i32(0), i32(NPAIR), pair_body, i32(0))
    wait_out(0)
    wait_out(1)


def kernel(X, hash_table):
    xt = X.astype(jnp.float32).T
    tv = (hash_table.astype(jnp.float32)
          .reshape(T // 128, 128, F).swapaxes(1, 2))
    rows_flat = _table_rows(tv)
    o = _grid_lookup(xt, rows_flat.reshape(T, F))
    return o.reshape(N // 128, F, 128).swapaxes(1, 2).reshape(N, F)


# speedup vs baseline: 4.7539x; 1.0043x over previous
import functools

import jax
import jax.numpy as jnp
from jax import lax
from jax.experimental import pallas as pl
from jax.experimental.pallas import tpu as pltpu
from jax.experimental.pallas import tpu_sc as plsc

N = 1048576
D = 3
T = 2097152
F = 8
RES = 101

P1 = -1640531535
P2 = 805459861

NW = 32
PTS = N // NW
P = 512
NG = P // 16
NCHUNK = PTS // P
NPAIR = NCHUNK // 2

NBLK = T // 128
BPW = NBLK // NW
BB = 16
NBATCH = BPW // BB


def _iota16():
    return lax.broadcasted_iota(jnp.int32, (16,), 0)


def _full16(v):
    return jnp.full((16,), v, jnp.int32)


_mesh = plsc.VectorSubcoreMesh(core_axis_name="c", subcore_axis_name="s")
_params = pltpu.CompilerParams(use_tc_tiling_on_sc=False,
                               needs_layout_passes=False)


@functools.partial(
    pl.kernel,
    mesh=_mesh,
    compiler_params=_params,
    out_type=jax.ShapeDtypeStruct((T * F,), jnp.float32),
    scratch_types=[
        pltpu.VMEM((BB * F, 128), jnp.float32),
        pltpu.VMEM((BB * F, 128), jnp.float32),
        pltpu.VMEM((BB * 128 * F,), jnp.float32),
        pltpu.VMEM((BB * 128 * F,), jnp.float32),
        pltpu.SemaphoreType.DMA,
        pltpu.SemaphoreType.DMA,
        pltpu.SemaphoreType.DMA,
        pltpu.SemaphoreType.DMA,
    ],
)
def _table_rows(tv_hbm, out_hbm, in0, in1, ou0, ou1, is0, is1, os0, os1):
    i32 = jnp.int32
    wid = lax.axis_index("s") * i32(2) + lax.axis_index("c")
    wbase = wid * i32(BPW)
    iot = _iota16()
    fpat = []
    cpat = []
    for k in range(8):
        m = iot + k * 16
        fpat.append(m & 7)
        cpat.append(lax.shift_right_logical(m, jnp.int32(3)))

    ins = (in0, in1)
    ous = (ou0, ou1)
    isems = (is0, is1)
    osems = (os0, os1)

    def transpose_batch(ib, ob):
        def blk_body(blk, c2):
            bv = _full16(blk * i32(8))
            for k in range(8):
                v = plsc.load_gather(ib, [bv + fpat[k], cpat[k]])
                ob[pl.ds(blk * i32(1024) + i32(k * 16), 16)] = v
            return c2

        lax.fori_loop(i32(0), i32(BB), blk_body, i32(0))

    def fire_in(bi, s):
        pltpu.async_copy(
            tv_hbm.at[pl.ds((wbase + bi * i32(BB)) * i32(F), BB * F)],
            ins[s], isems[s])

    def wait_in(s):
        pltpu.make_async_copy(tv_hbm.at[pl.ds(0, BB * F)], ins[s],
                              isems[s]).wait()

    def fire_out(bi, s):
        pltpu.async_copy(ous[s],
                         out_hbm.at[pl.ds((wbase + bi * i32(BB)) * i32(1024),
                                          BB * 1024)],
                         osems[s])

    def wait_out(s):
        pltpu.make_async_copy(ous[s], out_hbm.at[pl.ds(0, BB * 1024)],
                              osems[s]).wait()

    fire_in(i32(0), 0)

    def pair_body(t2, carry):
        bi0 = t2 * i32(2)
        fire_in(bi0 + 1, 1)
        wait_in(0)

        @pl.when(t2 > 0)
        def _():
            wait_out(0)

        transpose_batch(ins[0], ous[0])
        fire_out(bi0, 0)

        @pl.when(bi0 + 2 < NBATCH)
        def _():
            fire_in(bi0 + 2, 0)

        wait_in(1)

        @pl.when(t2 > 0)
        def _():
            wait_out(1)

        transpose_batch(ins[1], ous[1])
        fire_out(bi0 + 1, 1)
        return carry

    lax.fori_loop(i32(0), i32(NBATCH // 2), pair_body, i32(0))
    wait_out(0)
    wait_out(1)


@functools.partial(
    pl.kernel,
    mesh=_mesh,
    compiler_params=_params,
    out_type=jax.ShapeDtypeStruct((N * F,), jnp.float32),
    scratch_types=[
        pltpu.VMEM((3, P), jnp.float32),
        pltpu.VMEM((3, P), jnp.float32),
        pltpu.VMEM((NG, 128), jnp.int32),
        pltpu.VMEM((NG, 128), jnp.int32),
        pltpu.VMEM((8 * P, F), jnp.float32),
        pltpu.VMEM((8 * P, F), jnp.float32),
        pltpu.VMEM((P * F,), jnp.float32),
        pltpu.VMEM((P * F,), jnp.float32),
        pltpu.VMEM((3, 2 * P), jnp.float32),
        pltpu.SemaphoreType.DMA,
        pltpu.SemaphoreType.DMA,
        pltpu.SemaphoreType.DMA,
        pltpu.SemaphoreType.DMA,
    ],
)
def _grid_lookup(xt_hbm, table_hbm, out_hbm, wb0, wb1, ib0, ib1, rb0, rb1,
                 ob0, ob1, xb, g0, g1, o0, o1):
    i32 = jnp.int32
    wid = lax.axis_index("s") * i32(2) + lax.axis_index("c")
    base = wid * i32(PTS)
    iot = _iota16()

    wbs = (wb0, wb1)
    ibs = (ib0, ib1)
    rbs = (rb0, rb1)
    obs = (ob0, ob1)
    gsems = (g0, g1)
    osems = (o0, o1)

    def load_x_pair(t2):
        pltpu.sync_copy(xt_hbm.at[:, pl.ds(base + t2 * i32(2 * P), 2 * P)], xb)

    def fire_chunk(xoff, s):
        wb, ib, rb, gsem = wbs[s], ibs[s], rbs[s], gsems[s]

        def hash_group(g, c2):
            off = g * i32(16)
            ints = []
            for d in range(3):
                xs = (xb[d, pl.ds(xoff + off, 16)] + 1.0) / 2.0 * (RES - 1)
                ii = xs.astype(jnp.int32)
                wb[d, pl.ds(off, 16)] = xs - ii.astype(jnp.float32)
                ints.append(ii)
            ix, iy, iz = ints
            a0 = ix
            a1 = ix + 1
            b0 = iy * P1
            b1 = b0 + P1
            c0 = iz * P2
            c1 = c0 + P2
            for c in range(8):
                h = (a1 if c & 4 else a0) ^ (b1 if c & 2 else b0)
                h = (h ^ (c1 if c & 1 else c0)) & (T - 1)
                ib[g, pl.ds(c * 16, 16)] = h
            pltpu.async_copy(table_hbm.at[ib.at[g]],
                             rb.at[pl.ds(g * i32(128), 128)], gsem)
            return c2

        lax.fori_loop(i32(0), i32(NG), hash_group, i32(0))

    def drain_chunk(s):
        pltpu.make_async_copy(table_hbm.at[pl.ds(0, 8 * P)], rbs[s],
                              gsems[s]).wait()

    def wait_out(s):
        pltpu.make_async_copy(obs[s], out_hbm.at[pl.ds(0, P * F)],
                              osems[s]).wait()

    def interp_chunk(tc, s):
        wb, rb, ob, osem = wbs[s], rbs[s], obs[s], osems[s]

        def interp_group(g, c2):
            off = g * i32(16)
            wx = wb[0, pl.ds(off, 16)]
            wy = wb[1, pl.ds(off, 16)]
            wz = wb[2, pl.ds(off, 16)]
            ux = 1.0 - wx
            uy = 1.0 - wy
            uz = 1.0 - wz
            exy = [ux * uy, ux * wy, wx * uy, wx * wy]
            accs = [jnp.zeros((16,), jnp.float32) for _ in range(F)]
            rowbase = g * 128
            for c in range(8):
                wc = exy[c >> 1] * (wz if c & 1 else uz)
                ridx = _full16(rowbase + c * 16) + iot
                for f in range(F):
                    v = plsc.load_gather(rb, [ridx, _full16(f)])
                    accs[f] = accs[f] + wc * v
            obase = (lax.div(g, i32(8)) * i32(1024)
                     + lax.rem(g, i32(8)) * i32(16))
            for f in range(F):
                ob[pl.ds(obase + f * 128, 16)] = accs[f]
            return c2

        lax.fori_loop(i32(0), i32(NG), interp_group, i32(0))
        pltpu.async_copy(ob, out_hbm.at[pl.ds((base + tc * i32(P)) * 8,
                                              P * F)], osem)

    load_x_pair(i32(0))
    fire_chunk(i32(0), 0)

    def pair_body(t2, carry):
        tc0 = t2 * i32(2)
        fire_chunk(i32(P), 1)
        drain_chunk(0)

        @pl.when(t2 > 0)
        def _():
            wait_out(0)

        interp_chunk(tc0, 0)

        @pl.when(t2 + 1 < NPAIR)
        def _():
            load_x_pair(t2 + 1)
            fire_chunk(i32(0), 0)

        drain_chunk(1)

        @pl.when(t2 > 0)
        def _():
            wait_out(1)

        interp_chunk(tc0 + 1, 1)
        return carry

    lax.fori_loop(i32(0), i32(NPAIR), pair_body, i32(0))
    wait_out(0)
    wait_out(1)


def kernel(X, hash_table):
    xt = X.astype(jnp.float32).T
    tv = (hash_table.astype(jnp.float32)
          .reshape(T // 128, 128, F).swapaxes(1, 2))
    rows_flat = _table_rows(tv.reshape(T // 128 * F, 128))
    o = _grid_lookup(xt, rows_flat.reshape(T, F))
    return o.reshape(N // 128, F, 128).swapaxes(1, 2).reshape(N, F)
